# baseline (speedup 1.0000x reference)
import functools

import jax
import jax.numpy as jnp
from jax import lax
from jax.experimental import pallas as pl
from jax.experimental.pallas import tpu as pltpu
from jax.experimental.pallas import tpu_sc as plsc

N = 10000
E = 320000
D_IN = 128
H = 16
R = 237

LANES = 128
CHUNK_E = 1024
NCORES = 2
NSUB = 16
NTILES = NCORES * NSUB
E_PAD = 327680
EDGES_PER_TILE = E_PAD // NTILES
N_CHUNKS = EDGES_PER_TILE // CHUNK_E
ACC_ROWS = 10240
ZROWS = ACC_ROWS // NSUB


def _make_seg_sum(with_deg):
    mesh = plsc.VectorSubcoreMesh(core_axis_name="c", subcore_axis_name="s")
    out_type = [jax.ShapeDtypeStruct((NCORES, ACC_ROWS, H), jnp.float32)]
    if with_deg:
        out_type.append(jax.ShapeDtypeStruct((NCORES, ACC_ROWS, H), jnp.float32))
    scratch_types = [
        pltpu.VMEM((CHUNK_E,), jnp.int32),
        pltpu.VMEM((CHUNK_E,), jnp.int32),
        pltpu.VMEM((CHUNK_E,), jnp.int32),
        pltpu.VMEM((CHUNK_E,), jnp.int32),
        pltpu.VMEM((CHUNK_E, H), jnp.float32),
        pltpu.VMEM((CHUNK_E, H), jnp.float32),
        pltpu.VMEM((CHUNK_E, H), jnp.float32),
        pltpu.VMEM_SHARED((ACC_ROWS, H), jnp.float32),
        pltpu.VMEM_SHARED((ACC_ROWS, H), jnp.float32),
        pltpu.SemaphoreType.DMA,
        pltpu.SemaphoreType.DMA,
    ]

    def body(p_hbm, src_hbm, dst_hbm, zeros_hbm, ones_hbm, *rest):
        if with_deg:
            acc_out, deg_out = rest[0], rest[1]
            scr = rest[2:]
        else:
            acc_out = rest[0]
            deg_out = None
            scr = rest[1:]
        (src_a, src_b, dst_a, dst_b, rows_a, rows_b, ones_v,
         acc_s, dacc_s, sem_g, sem_s) = scr
        cid = lax.axis_index("c")
        sid = lax.axis_index("s")
        wid = cid * NSUB + sid
        e0 = wid * EDGES_PER_TILE

        def load_and_fire(src_v, dst_v, rows_v, g):
            base = e0 + g * CHUNK_E
            pltpu.sync_copy(src_hbm.at[pl.ds(base, CHUNK_E)], src_v)
            pltpu.sync_copy(dst_hbm.at[pl.ds(base, CHUNK_E)], dst_v)
            pltpu.async_copy(p_hbm.at[src_v], rows_v, sem_g)

        def drain_gather(src_v, rows_v):
            pltpu.make_async_copy(p_hbm.at[src_v], rows_v, sem_g).wait()

        def fire_scatter(rows_v, dst_v):
            pltpu.async_copy(rows_v, acc_s.at[dst_v], sem_s, add=True)
            if with_deg:
                pltpu.async_copy(ones_v, dacc_s.at[dst_v], sem_s, add=True)

        def drain_scatter(rows_v, dst_v):
            pltpu.make_async_copy(rows_v, acc_s.at[dst_v], sem_s).wait()
            if with_deg:
                pltpu.make_async_copy(ones_v, dacc_s.at[dst_v], sem_s).wait()

        load_and_fire(src_a, dst_a, rows_a, 0)
        if with_deg:
            pltpu.sync_copy(ones_hbm, ones_v)
        z0 = sid * ZROWS
        pltpu.sync_copy(zeros_hbm, acc_s.at[pl.ds(z0, ZROWS)])
        if with_deg:
            pltpu.sync_copy(zeros_hbm, dacc_s.at[pl.ds(z0, ZROWS)])
        plsc.subcore_barrier()

        def step(g2, carry):
            c = g2 * 2
            drain_gather(src_a, rows_a)

            @pl.when(c > 0)
            def _():
                drain_scatter(rows_b, dst_b)

            load_and_fire(src_b, dst_b, rows_b, c + 1)
            fire_scatter(rows_a, dst_a)
            drain_gather(src_b, rows_b)
            drain_scatter(rows_a, dst_a)

            @pl.when(c + 2 < N_CHUNKS)
            def _():
                load_and_fire(src_a, dst_a, rows_a, c + 2)

            fire_scatter(rows_b, dst_b)
            return carry

        lax.fori_loop(0, N_CHUNKS // 2, step, 0)
        drain_scatter(rows_b, dst_b)
        plsc.subcore_barrier()

        pltpu.sync_copy(acc_s.at[pl.ds(z0, ZROWS)],
                        acc_out.at[cid, pl.ds(z0, ZROWS)])
        if with_deg:
            pltpu.sync_copy(dacc_s.at[pl.ds(z0, ZROWS)],
                            deg_out.at[cid, pl.ds(z0, ZROWS)])

    return pl.kernel(body, out_type=out_type, mesh=mesh,
                     scratch_types=scratch_types,
                     compiler_params=pltpu.CompilerParams(
                         use_tc_tiling_on_sc=False))


_seg_sum_deg = _make_seg_sum(True)
_seg_sum = _make_seg_sum(False)


BN = 2000


def _dotT(a, w):
    return lax.dot_general(a, w, (((1,), (1,)), ((), ())),
                           preferred_element_type=jnp.float32)


def _stage1_body(x_ref, wl_ref, wr_ref, p_ref, q_ref):
    xb = x_ref[...]
    p_ref[...] = _dotT(xb, wl_ref[...])
    q_ref[...] = _dotT(xb, wr_ref[...])


def _tc_stage1(x, wl, wr):
    return pl.pallas_call(
        _stage1_body,
        grid=(N // BN,),
        in_specs=[
            pl.BlockSpec((BN, D_IN), lambda i: (i, 0)),
            pl.BlockSpec((H, D_IN), lambda i: (0, 0)),
            pl.BlockSpec((H, D_IN), lambda i: (0, 0)),
        ],
        out_specs=[pl.BlockSpec((BN, H), lambda i: (i, 0))] * 2,
        out_shape=[jax.ShapeDtypeStruct((N, H), jnp.float32)] * 2,
    )(x, wl, wr)


def _mid_first_body(a0_ref, a1_ref, d0_ref, d1_ref, q_ref, b_ref, wl_ref,
                    wr_ref, p2_ref, q2_ref, rdeg_ref):
    rdeg = 1.0 / jnp.maximum(d0_ref[...] + d1_ref[...], 1.0)
    h = jnp.maximum((a0_ref[...] + a1_ref[...]) * rdeg + b_ref[...] + q_ref[...], 0.0)
    p2_ref[...] = _dotT(h, wl_ref[...])
    q2_ref[...] = _dotT(h, wr_ref[...])
    rdeg_ref[...] = rdeg


def _tc_mid_first(a0, a1, d0, d1, q, b, wl, wr):
    nh = pl.BlockSpec((BN, H), lambda i: (i, 0))
    wspec = pl.BlockSpec((H, H), lambda i: (0, 0))
    return pl.pallas_call(
        _mid_first_body,
        grid=(N // BN,),
        in_specs=[nh, nh, nh, nh, nh, pl.BlockSpec((1, H), lambda i: (0, 0)),
                  wspec, wspec],
        out_specs=[nh, nh, nh],
        out_shape=[jax.ShapeDtypeStruct((N, H), jnp.float32)] * 3,
    )(a0, a1, d0, d1, q, b, wl, wr)


def _mid_body(a0_ref, a1_ref, rdeg_ref, q_ref, b_ref, wl_ref, wr_ref,
              p3_ref, q3_ref):
    h = jnp.maximum((a0_ref[...] + a1_ref[...]) * rdeg_ref[...]
                    + b_ref[...] + q_ref[...], 0.0)
    p3_ref[...] = _dotT(h, wl_ref[...])
    q3_ref[...] = _dotT(h, wr_ref[...])


def _tc_mid(a0, a1, rdeg, q, b, wl, wr):
    nh = pl.BlockSpec((BN, H), lambda i: (i, 0))
    wspec = pl.BlockSpec((H, H), lambda i: (0, 0))
    return pl.pallas_call(
        _mid_body,
        grid=(N // BN,),
        in_specs=[nh, nh, nh, nh, pl.BlockSpec((1, H), lambda i: (0, 0)),
                  wspec, wspec],
        out_specs=[nh, nh],
        out_shape=[jax.ShapeDtypeStruct((N, H), jnp.float32)] * 2,
    )(a0, a1, rdeg, q, b, wl, wr)


def _tail_body(a0_ref, a1_ref, rdeg_ref, q_ref, b_ref, wfc_ref, bfc_ref,
               out_ref):
    h = (a0_ref[...] + a1_ref[...]) * rdeg_ref[...] + b_ref[...] + q_ref[...]
    out_ref[...] = _dotT(h, wfc_ref[...]) + bfc_ref[...]


def _tc_tail(a0, a1, rdeg, q, b, wfc, bfc):
    nh = pl.BlockSpec((BN, H), lambda i: (i, 0))
    return pl.pallas_call(
        _tail_body,
        grid=(N // BN,),
        in_specs=[nh, nh, nh, nh, pl.BlockSpec((1, H), lambda i: (0, 0)),
                  pl.BlockSpec((R, H), lambda i: (0, 0)),
                  pl.BlockSpec((1, R), lambda i: (0, 0))],
        out_specs=pl.BlockSpec((BN, R), lambda i: (i, 0)),
        out_shape=jax.ShapeDtypeStruct((N, R), jnp.float32),
    )(a0, a1, rdeg, q, b, wfc, bfc)


@jax.jit
def kernel(x, edge_index, W1l, b1l, W1r, W2l, b2l, W2r, W3l, b3l, W3r, Wfc, bfc):
    pad = E_PAD - E
    src2d = jnp.concatenate([edge_index[0], jnp.zeros((pad,), jnp.int32)])
    dst2d = jnp.concatenate([edge_index[1], jnp.full((pad,), N, jnp.int32)])
    zeros_c = jnp.zeros((ZROWS, H), jnp.float32)
    ones_c = jnp.ones((CHUNK_E, H), jnp.float32)

    p1, q1 = _tc_stage1(x, W1l, W1r)
    acc1, deg = _seg_sum_deg(p1, src2d, dst2d, zeros_c, ones_c)
    p2, q2, rdeg = _tc_mid_first(acc1[0, :N], acc1[1, :N], deg[0, :N],
                                 deg[1, :N], q1, b1l.reshape(1, H), W2l, W2r)
    acc2 = _seg_sum(p2, src2d, dst2d, zeros_c, ones_c)[0]
    p3, q3 = _tc_mid(acc2[0, :N], acc2[1, :N], rdeg, q2, b2l.reshape(1, H),
                     W3l, W3r)
    acc3 = _seg_sum(p3, src2d, dst2d, zeros_c, ones_c)[0]
    return _tc_tail(acc3[0, :N], acc3[1, :N], rdeg, q3, b3l.reshape(1, H),
                    Wfc, bfc.reshape(1, R))


# speedup vs baseline: 1.3341x; 1.3341x over previous
import functools

import jax
import jax.numpy as jnp
from jax import lax
from jax.experimental import pallas as pl
from jax.experimental.pallas import tpu as pltpu
from jax.experimental.pallas import tpu_sc as plsc

N = 10000
E = 320000
D_IN = 128
H = 16
R = 237

LANES = 128
CHUNK_E = 1024
NCORES = 2
NSUB = 16
NTILES = NCORES * NSUB
E_PAD = 327680
EDGES_PER_TILE = E_PAD // NTILES
N_CHUNKS = EDGES_PER_TILE // CHUNK_E
ACC_ROWS = 10240
ZROWS = ACC_ROWS // NSUB


def _make_seg_sum(with_deg):
    mesh = plsc.VectorSubcoreMesh(core_axis_name="c", subcore_axis_name="s")
    out_type = [jax.ShapeDtypeStruct((NCORES, ACC_ROWS, H), jnp.float32)]
    if with_deg:
        out_type.append(jax.ShapeDtypeStruct((NCORES, ACC_ROWS, H), jnp.float32))
    scratch_types = [
        pltpu.VMEM((CHUNK_E,), jnp.int32),
        pltpu.VMEM((CHUNK_E,), jnp.int32),
        pltpu.VMEM((CHUNK_E,), jnp.int32),
        pltpu.VMEM((CHUNK_E,), jnp.int32),
        pltpu.VMEM((CHUNK_E, H), jnp.float32),
        pltpu.VMEM((CHUNK_E, H), jnp.float32),
        pltpu.VMEM((CHUNK_E, H), jnp.float32),
        pltpu.VMEM_SHARED((N, H), jnp.float32),
        pltpu.VMEM_SHARED((ACC_ROWS, H), jnp.float32),
        pltpu.VMEM_SHARED((ACC_ROWS, H), jnp.float32),
        pltpu.SemaphoreType.DMA,
        pltpu.SemaphoreType.DMA,
    ]

    def body(p_hbm, src_hbm, dst_hbm, zeros_hbm, ones_hbm, *rest):
        if with_deg:
            acc_out, deg_out = rest[0], rest[1]
            scr = rest[2:]
        else:
            acc_out = rest[0]
            deg_out = None
            scr = rest[1:]
        (src_a, src_b, dst_a, dst_b, rows_a, rows_b, ones_v,
         p_s, acc_s, dacc_s, sem_g, sem_s) = scr
        cid = lax.axis_index("c")
        sid = lax.axis_index("s")
        wid = cid * NSUB + sid
        e0 = wid * EDGES_PER_TILE

        def load_idx(src_v, dst_v, g):
            base = e0 + g * CHUNK_E
            pltpu.sync_copy(src_hbm.at[pl.ds(base, CHUNK_E)], src_v)
            pltpu.sync_copy(dst_hbm.at[pl.ds(base, CHUNK_E)], dst_v)

        def fire_gather(src_v, rows_v):
            pltpu.async_copy(p_s.at[src_v], rows_v, sem_g)

        def load_and_fire(src_v, dst_v, rows_v, g):
            load_idx(src_v, dst_v, g)
            fire_gather(src_v, rows_v)

        def drain_gather(src_v, rows_v):
            pltpu.make_async_copy(p_s.at[src_v], rows_v, sem_g).wait()

        def fire_scatter(rows_v, dst_v):
            pltpu.async_copy(rows_v, acc_s.at[dst_v], sem_s, add=True)
            if with_deg:
                pltpu.async_copy(ones_v, dacc_s.at[dst_v], sem_s, add=True)

        def drain_scatter(rows_v, dst_v):
            pltpu.make_async_copy(rows_v, acc_s.at[dst_v], sem_s).wait()
            if with_deg:
                pltpu.make_async_copy(ones_v, dacc_s.at[dst_v], sem_s).wait()

        load_idx(src_a, dst_a, 0)
        if with_deg:
            pltpu.sync_copy(ones_hbm, ones_v)

        @pl.when(sid < NSUB - 1)
        def _():
            pltpu.sync_copy(p_hbm.at[pl.ds(sid * ZROWS, ZROWS)],
                            p_s.at[pl.ds(sid * ZROWS, ZROWS)])

        @pl.when(sid == NSUB - 1)
        def _():
            pltpu.sync_copy(p_hbm.at[pl.ds((NSUB - 1) * ZROWS, N - (NSUB - 1) * ZROWS)],
                            p_s.at[pl.ds((NSUB - 1) * ZROWS, N - (NSUB - 1) * ZROWS)])

        z0 = sid * ZROWS
        pltpu.sync_copy(zeros_hbm, acc_s.at[pl.ds(z0, ZROWS)])
        if with_deg:
            pltpu.sync_copy(zeros_hbm, dacc_s.at[pl.ds(z0, ZROWS)])
        plsc.subcore_barrier()
        fire_gather(src_a, rows_a)

        def step(g2, carry):
            c = g2 * 2
            drain_gather(src_a, rows_a)

            @pl.when(c > 0)
            def _():
                drain_scatter(rows_b, dst_b)

            load_and_fire(src_b, dst_b, rows_b, c + 1)
            fire_scatter(rows_a, dst_a)
            drain_gather(src_b, rows_b)
            drain_scatter(rows_a, dst_a)

            @pl.when(c + 2 < N_CHUNKS)
            def _():
                load_and_fire(src_a, dst_a, rows_a, c + 2)

            fire_scatter(rows_b, dst_b)
            return carry

        lax.fori_loop(0, N_CHUNKS // 2, step, 0)
        drain_scatter(rows_b, dst_b)
        plsc.subcore_barrier()

        pltpu.sync_copy(acc_s.at[pl.ds(z0, ZROWS)],
                        acc_out.at[cid, pl.ds(z0, ZROWS)])
        if with_deg:
            pltpu.sync_copy(dacc_s.at[pl.ds(z0, ZROWS)],
                            deg_out.at[cid, pl.ds(z0, ZROWS)])

    return pl.kernel(body, out_type=out_type, mesh=mesh,
                     scratch_types=scratch_types,
                     compiler_params=pltpu.CompilerParams(
                         use_tc_tiling_on_sc=False))


_seg_sum_deg = _make_seg_sum(True)
_seg_sum = _make_seg_sum(False)


BN = 2000


def _dotT(a, w):
    return lax.dot_general(a, w, (((1,), (1,)), ((), ())),
                           preferred_element_type=jnp.float32)


def _stage1_body(x_ref, wl_ref, wr_ref, p_ref, q_ref):
    xb = x_ref[...]
    p_ref[...] = _dotT(xb, wl_ref[...])
    q_ref[...] = _dotT(xb, wr_ref[...])


def _tc_stage1(x, wl, wr):
    return pl.pallas_call(
        _stage1_body,
        grid=(N // BN,),
        in_specs=[
            pl.BlockSpec((BN, D_IN), lambda i: (i, 0)),
            pl.BlockSpec((H, D_IN), lambda i: (0, 0)),
            pl.BlockSpec((H, D_IN), lambda i: (0, 0)),
        ],
        out_specs=[pl.BlockSpec((BN, H), lambda i: (i, 0))] * 2,
        out_shape=[jax.ShapeDtypeStruct((N, H), jnp.float32)] * 2,
    )(x, wl, wr)


def _mid_first_body(a0_ref, a1_ref, d0_ref, d1_ref, q_ref, b_ref, wl_ref,
                    wr_ref, p2_ref, q2_ref, rdeg_ref):
    rdeg = 1.0 / jnp.maximum(d0_ref[...] + d1_ref[...], 1.0)
    h = jnp.maximum((a0_ref[...] + a1_ref[...]) * rdeg + b_ref[...] + q_ref[...], 0.0)
    p2_ref[...] = _dotT(h, wl_ref[...])
    q2_ref[...] = _dotT(h, wr_ref[...])
    rdeg_ref[...] = rdeg


def _tc_mid_first(a0, a1, d0, d1, q, b, wl, wr):
    nh = pl.BlockSpec((BN, H), lambda i: (i, 0))
    wspec = pl.BlockSpec((H, H), lambda i: (0, 0))
    return pl.pallas_call(
        _mid_first_body,
        grid=(N // BN,),
        in_specs=[nh, nh, nh, nh, nh, pl.BlockSpec((1, H), lambda i: (0, 0)),
                  wspec, wspec],
        out_specs=[nh, nh, nh],
        out_shape=[jax.ShapeDtypeStruct((N, H), jnp.float32)] * 3,
    )(a0, a1, d0, d1, q, b, wl, wr)


def _mid_body(a0_ref, a1_ref, rdeg_ref, q_ref, b_ref, wl_ref, wr_ref,
              p3_ref, q3_ref):
    h = jnp.maximum((a0_ref[...] + a1_ref[...]) * rdeg_ref[...]
                    + b_ref[...] + q_ref[...], 0.0)
    p3_ref[...] = _dotT(h, wl_ref[...])
    q3_ref[...] = _dotT(h, wr_ref[...])


def _tc_mid(a0, a1, rdeg, q, b, wl, wr):
    nh = pl.BlockSpec((BN, H), lambda i: (i, 0))
    wspec = pl.BlockSpec((H, H), lambda i: (0, 0))
    return pl.pallas_call(
        _mid_body,
        grid=(N // BN,),
        in_specs=[nh, nh, nh, nh, pl.BlockSpec((1, H), lambda i: (0, 0)),
                  wspec, wspec],
        out_specs=[nh, nh],
        out_shape=[jax.ShapeDtypeStruct((N, H), jnp.float32)] * 2,
    )(a0, a1, rdeg, q, b, wl, wr)


def _tail_body(a0_ref, a1_ref, rdeg_ref, q_ref, b_ref, wfc_ref, bfc_ref,
               out_ref):
    h = (a0_ref[...] + a1_ref[...]) * rdeg_ref[...] + b_ref[...] + q_ref[...]
    out_ref[...] = _dotT(h, wfc_ref[...]) + bfc_ref[...]


def _tc_tail(a0, a1, rdeg, q, b, wfc, bfc):
    nh = pl.BlockSpec((BN, H), lambda i: (i, 0))
    return pl.pallas_call(
        _tail_body,
        grid=(N // BN,),
        in_specs=[nh, nh, nh, nh, pl.BlockSpec((1, H), lambda i: (0, 0)),
                  pl.BlockSpec((R, H), lambda i: (0, 0)),
                  pl.BlockSpec((1, R), lambda i: (0, 0))],
        out_specs=pl.BlockSpec((BN, R), lambda i: (i, 0)),
        out_shape=jax.ShapeDtypeStruct((N, R), jnp.float32),
    )(a0, a1, rdeg, q, b, wfc, bfc)


@jax.jit
def kernel(x, edge_index, W1l, b1l, W1r, W2l, b2l, W2r, W3l, b3l, W3r, Wfc, bfc):
    pad = E_PAD - E
    src2d = jnp.concatenate([edge_index[0], jnp.zeros((pad,), jnp.int32)])
    dst2d = jnp.concatenate([edge_index[1], jnp.full((pad,), N, jnp.int32)])
    zeros_c = jnp.zeros((ZROWS, H), jnp.float32)
    ones_c = jnp.ones((CHUNK_E, H), jnp.float32)

    p1, q1 = _tc_stage1(x, W1l, W1r)
    acc1, deg = _seg_sum_deg(p1, src2d, dst2d, zeros_c, ones_c)
    p2, q2, rdeg = _tc_mid_first(acc1[0, :N], acc1[1, :N], deg[0, :N],
                                 deg[1, :N], q1, b1l.reshape(1, H), W2l, W2r)
    acc2 = _seg_sum(p2, src2d, dst2d, zeros_c, ones_c)[0]
    p3, q3 = _tc_mid(acc2[0, :N], acc2[1, :N], rdeg, q2, b2l.reshape(1, H),
                     W3l, W3r)
    acc3 = _seg_sum(p3, src2d, dst2d, zeros_c, ones_c)[0]
    return _tc_tail(acc3[0, :N], acc3[1, :N], rdeg, q3, b3l.reshape(1, H),
                    Wfc, bfc.reshape(1, R))


# speedup vs baseline: 1.9162x; 1.4364x over previous
import functools

import jax
import jax.numpy as jnp
from jax import lax
from jax.experimental import pallas as pl
from jax.experimental.pallas import tpu as pltpu
from jax.experimental.pallas import tpu_sc as plsc

N = 10000
E = 320000
D_IN = 128
H = 16
R = 237

LANES = 128
CHUNK_E = 1024
NCORES = 2
NSUB = 16
NTILES = NCORES * NSUB
E_PAD = 327680
EDGES_PER_TILE = E_PAD // NTILES
N_CHUNKS = EDGES_PER_TILE // CHUNK_E
ACC_ROWS = 10240
ZROWS = ACC_ROWS // NSUB


def _make_seg_sum(with_deg):
    mesh = plsc.VectorSubcoreMesh(core_axis_name="c", subcore_axis_name="s")
    out_type = [jax.ShapeDtypeStruct((NCORES, ACC_ROWS, H), jnp.float32)]
    if with_deg:
        out_type.append(jax.ShapeDtypeStruct((NCORES, ACC_ROWS, H), jnp.float32))
    scratch_types = [
        pltpu.VMEM((CHUNK_E,), jnp.int32),
        pltpu.VMEM((CHUNK_E,), jnp.int32),
        pltpu.VMEM((CHUNK_E,), jnp.int32),
        pltpu.VMEM((CHUNK_E,), jnp.int32),
        pltpu.VMEM((CHUNK_E, H), jnp.float32),
        pltpu.VMEM((CHUNK_E, H), jnp.float32),
        pltpu.VMEM((CHUNK_E, H), jnp.float32),
        pltpu.VMEM_SHARED((ACC_ROWS, H), jnp.float32),
        pltpu.VMEM_SHARED((ACC_ROWS, H), jnp.float32),
        pltpu.VMEM_SHARED((ACC_ROWS, H), jnp.float32),
        pltpu.SemaphoreType.DMA,
        pltpu.SemaphoreType.DMA,
    ]

    def body(p_hbm, src_hbm, dst_hbm, zeros_hbm, ones_hbm, *rest):
        if with_deg:
            acc_out, deg_out = rest[0], rest[1]
            scr = rest[2:]
        else:
            acc_out = rest[0]
            deg_out = None
            scr = rest[1:]
        (src_a, src_b, dst_a, dst_b, rows_a, rows_b, ones_v,
         p_s, acc_s, dacc_s, sem_g, sem_s) = scr
        cid = lax.axis_index("c")
        sid = lax.axis_index("s")
        wid = cid * NSUB + sid
        e0 = wid * EDGES_PER_TILE

        def load_idx(src_v, dst_v, g):
            base = e0 + g * CHUNK_E
            pltpu.sync_copy(src_hbm.at[pl.ds(base, CHUNK_E)], src_v)
            pltpu.sync_copy(dst_hbm.at[pl.ds(base, CHUNK_E)], dst_v)

        def fire_gather(src_v, rows_v):
            pltpu.async_copy(p_s.at[src_v], rows_v, sem_g)

        def load_and_fire(src_v, dst_v, rows_v, g):
            load_idx(src_v, dst_v, g)
            fire_gather(src_v, rows_v)

        def drain_gather(src_v, rows_v):
            pltpu.make_async_copy(p_s.at[src_v], rows_v, sem_g).wait()

        def fire_scatter(rows_v, dst_v):
            pltpu.async_copy(rows_v, acc_s.at[dst_v], sem_s, add=True)
            if with_deg:
                pltpu.async_copy(ones_v, dacc_s.at[dst_v], sem_s, add=True)

        def drain_scatter(rows_v, dst_v):
            pltpu.make_async_copy(rows_v, acc_s.at[dst_v], sem_s).wait()
            if with_deg:
                pltpu.make_async_copy(ones_v, dacc_s.at[dst_v], sem_s).wait()

        load_idx(src_a, dst_a, 0)
        if with_deg:
            pltpu.sync_copy(ones_hbm, ones_v)

        z0 = sid * ZROWS
        pltpu.sync_copy(p_hbm.at[pl.ds(z0, ZROWS)], p_s.at[pl.ds(z0, ZROWS)])
        pltpu.sync_copy(zeros_hbm, acc_s.at[pl.ds(z0, ZROWS)])
        if with_deg:
            pltpu.sync_copy(zeros_hbm, dacc_s.at[pl.ds(z0, ZROWS)])
        plsc.subcore_barrier()
        fire_gather(src_a, rows_a)

        def step(g2, carry):
            c = g2 * 2
            drain_gather(src_a, rows_a)

            @pl.when(c > 0)
            def _():
                drain_scatter(rows_b, dst_b)

            load_and_fire(src_b, dst_b, rows_b, c + 1)
            fire_scatter(rows_a, dst_a)
            drain_gather(src_b, rows_b)
            drain_scatter(rows_a, dst_a)

            @pl.when(c + 2 < N_CHUNKS)
            def _():
                load_and_fire(src_a, dst_a, rows_a, c + 2)

            fire_scatter(rows_b, dst_b)
            return carry

        lax.fori_loop(0, N_CHUNKS // 2, step, 0)
        drain_scatter(rows_b, dst_b)
        plsc.subcore_barrier()

        pltpu.sync_copy(acc_s.at[pl.ds(z0, ZROWS)],
                        acc_out.at[cid, pl.ds(z0, ZROWS)])
        if with_deg:
            pltpu.sync_copy(dacc_s.at[pl.ds(z0, ZROWS)],
                            deg_out.at[cid, pl.ds(z0, ZROWS)])

    return pl.kernel(body, out_type=out_type, mesh=mesh,
                     scratch_types=scratch_types,
                     compiler_params=pltpu.CompilerParams(
                         use_tc_tiling_on_sc=False))


_seg_sum_deg = _make_seg_sum(True)
_seg_sum = _make_seg_sum(False)


NP = ACC_ROWS // 8
BP = NP // 5
BNODE = BP * 8


def _dotT(a, w):
    return lax.dot_general(a, w, (((1,), (1,)), ((), ())),
                           preferred_element_type=jnp.float32)


def _stage1_body(x_ref, wl_ref, wr_ref, p_ref, q_ref):
    xb = x_ref[...]
    wl = wl_ref[...]
    wr = wr_ref[...]
    p_ref[...] = jnp.concatenate(
        [_dotT(xb[:, a, :], wl) for a in range(8)], axis=1)
    q_ref[...] = jnp.concatenate(
        [_dotT(xb[:, a, :], wr) for a in range(8)], axis=1)


def _tc_stage1(x3, wl, wr):
    return pl.pallas_call(
        _stage1_body,
        grid=(5,),
        in_specs=[
            pl.BlockSpec((BP, 8, D_IN), lambda i: (i, 0, 0)),
            pl.BlockSpec((H, D_IN), lambda i: (0, 0)),
            pl.BlockSpec((H, D_IN), lambda i: (0, 0)),
        ],
        out_specs=[pl.BlockSpec((BP, 128), lambda i: (i, 0))] * 2,
        out_shape=[jax.ShapeDtypeStruct((NP, 128), jnp.float32)] * 2,
    )(x3, wl, wr)


_PK = pl.BlockSpec((BP, 128), lambda i: (i, 0))
_A0 = pl.BlockSpec((1, BP, 128), lambda i: (0, i, 0))
_A1 = pl.BlockSpec((1, BP, 128), lambda i: (1, i, 0))
_B = pl.BlockSpec((1, 128), lambda i: (0, 0))
_K = pl.BlockSpec((128, 128), lambda i: (0, 0))


def _mid_first_body(a0_ref, a1_ref, d0_ref, d1_ref, q_ref, b_ref, kl_ref,
                    kr_ref, p2_ref, q2_ref, rdeg_ref):
    rdeg = 1.0 / jnp.maximum(d0_ref[0] + d1_ref[0], 1.0)
    h = jnp.maximum((a0_ref[0] + a1_ref[0]) * rdeg + b_ref[...] + q_ref[...],
                    0.0)
    p2_ref[...] = jnp.dot(h, kl_ref[...], preferred_element_type=jnp.float32)
    q2_ref[...] = jnp.dot(h, kr_ref[...], preferred_element_type=jnp.float32)
    rdeg_ref[...] = rdeg


def _tc_mid_first(accp, degp, q, b, kl, kr):
    return pl.pallas_call(
        _mid_first_body,
        grid=(5,),
        in_specs=[_A0, _A1, _A0, _A1, _PK, _B, _K, _K],
        out_specs=[_PK, _PK, _PK],
        out_shape=[jax.ShapeDtypeStruct((NP, 128), jnp.float32)] * 3,
    )(accp, accp, degp, degp, q, b, kl, kr)


def _mid_body(a0_ref, a1_ref, rdeg_ref, q_ref, b_ref, kl_ref, kr_ref,
              p3_ref, q3_ref):
    h = jnp.maximum((a0_ref[0] + a1_ref[0]) * rdeg_ref[...]
                    + b_ref[...] + q_ref[...], 0.0)
    p3_ref[...] = jnp.dot(h, kl_ref[...], preferred_element_type=jnp.float32)
    q3_ref[...] = jnp.dot(h, kr_ref[...], preferred_element_type=jnp.float32)


def _tc_mid(accp, rdeg, q, b, kl, kr):
    return pl.pallas_call(
        _mid_body,
        grid=(5,),
        in_specs=[_A0, _A1, _PK, _PK, _B, _K, _K],
        out_specs=[_PK, _PK],
        out_shape=[jax.ShapeDtypeStruct((NP, 128), jnp.float32)] * 2,
    )(accp, accp, rdeg, q, b, kl, kr)


def _h3_body(a0_ref, a1_ref, rdeg_ref, q_ref, b_ref, h_ref):
    h_ref[...] = ((a0_ref[0] + a1_ref[0]) * rdeg_ref[...]
                  + b_ref[...] + q_ref[...])


def _tc_h3(accp, rdeg, q, b):
    return pl.pallas_call(
        _h3_body,
        grid=(5,),
        in_specs=[_A0, _A1, _PK, _PK, _B],
        out_specs=_PK,
        out_shape=jax.ShapeDtypeStruct((NP, 128), jnp.float32),
    )(accp, accp, rdeg, q, b)


def _fc_body(h_ref, wfc_ref, bfc_ref, out_ref):
    out_ref[...] = _dotT(h_ref[...], wfc_ref[...]) + bfc_ref[...]


def _tc_fc(hu, wfc, bfc):
    return pl.pallas_call(
        _fc_body,
        grid=(5,),
        in_specs=[pl.BlockSpec((BNODE, H), lambda i: (i, 0)),
                  pl.BlockSpec((R, H), lambda i: (0, 0)),
                  pl.BlockSpec((1, R), lambda i: (0, 0))],
        out_specs=pl.BlockSpec((BNODE, R), lambda i: (i, 0)),
        out_shape=jax.ShapeDtypeStruct((N, R), jnp.float32),
    )(hu, wfc, bfc)


@jax.jit
def kernel(x, edge_index, W1l, b1l, W1r, W2l, b2l, W2r, W3l, b3l, W3r, Wfc, bfc):
    pad = E_PAD - E
    src1d = jnp.concatenate([edge_index[0], jnp.zeros((pad,), jnp.int32)])
    dst1d = jnp.concatenate([edge_index[1], jnp.full((pad,), N, jnp.int32)])
    zeros_c = jnp.zeros((ZROWS, H), jnp.float32)
    ones_c = jnp.ones((CHUNK_E, H), jnp.float32)
    eye8 = jnp.eye(8, dtype=jnp.float32)
    k2l, k2r = jnp.kron(eye8, W2l.T), jnp.kron(eye8, W2r.T)
    k3l, k3r = jnp.kron(eye8, W3l.T), jnp.kron(eye8, W3r.T)
    b1p = jnp.tile(b1l, 8).reshape(1, 128)
    b2p = jnp.tile(b2l, 8).reshape(1, 128)
    b3p = jnp.tile(b3l, 8).reshape(1, 128)

    p1, q1 = _tc_stage1(x.reshape(N // 8, 8, D_IN), W1l, W1r)
    acc1, deg = _seg_sum_deg(p1.reshape(ACC_ROWS, H), src1d, dst1d,
                             zeros_c, ones_c)
    p2, q2, rdeg = _tc_mid_first(acc1.reshape(NCORES, NP, 128),
                                 deg.reshape(NCORES, NP, 128),
                                 q1, b1p, k2l, k2r)
    acc2 = _seg_sum(p2.reshape(ACC_ROWS, H), src1d, dst1d, zeros_c, ones_c)[0]
    p3, q3 = _tc_mid(acc2.reshape(NCORES, NP, 128), rdeg, q2, b2p, k3l, k3r)
    acc3 = _seg_sum(p3.reshape(ACC_ROWS, H), src1d, dst1d, zeros_c, ones_c)[0]
    h3 = _tc_h3(acc3.reshape(NCORES, NP, 128), rdeg, q3, b3p)
    return _tc_fc(h3.reshape(ACC_ROWS, H), Wfc, bfc.reshape(1, R))


# speedup vs baseline: 1.9359x; 1.0102x over previous
import functools

import jax
import jax.numpy as jnp
from jax import lax
from jax.experimental import pallas as pl
from jax.experimental.pallas import tpu as pltpu
from jax.experimental.pallas import tpu_sc as plsc

N = 10000
E = 320000
D_IN = 128
H = 16
R = 237

LANES = 128
CHUNK_E = 1024
NCORES = 2
NSUB = 16
NTILES = NCORES * NSUB
E_PAD = 327680
EDGES_PER_TILE = E_PAD // NTILES
N_CHUNKS = EDGES_PER_TILE // CHUNK_E
ACC_ROWS = 10240
ZROWS = ACC_ROWS // NSUB


def _make_seg_sum(with_deg):
    mesh = plsc.VectorSubcoreMesh(core_axis_name="c", subcore_axis_name="s")
    out_type = [jax.ShapeDtypeStruct((NCORES, ACC_ROWS, H), jnp.float32)]
    if with_deg:
        out_type.append(jax.ShapeDtypeStruct((NCORES, ACC_ROWS), jnp.float32))
    scratch_types = [
        pltpu.VMEM((CHUNK_E,), jnp.int32),
        pltpu.VMEM((CHUNK_E,), jnp.int32),
        pltpu.VMEM((CHUNK_E,), jnp.int32),
        pltpu.VMEM((CHUNK_E,), jnp.int32),
        pltpu.VMEM((CHUNK_E, H), jnp.float32),
        pltpu.VMEM((CHUNK_E, H), jnp.float32),
        pltpu.VMEM((CHUNK_E,), jnp.float32),
        pltpu.VMEM_SHARED((ACC_ROWS, H), jnp.float32),
        pltpu.VMEM_SHARED((ACC_ROWS, H), jnp.float32),
        pltpu.VMEM_SHARED((ACC_ROWS,), jnp.float32),
        pltpu.SemaphoreType.DMA,
        pltpu.SemaphoreType.DMA,
    ]

    def body(p_hbm, src_hbm, dst_hbm, zeros_hbm, zeros1_hbm, ones_hbm, *rest):
        if with_deg:
            acc_out, deg_out = rest[0], rest[1]
            scr = rest[2:]
        else:
            acc_out = rest[0]
            deg_out = None
            scr = rest[1:]
        (src_a, src_b, dst_a, dst_b, rows_a, rows_b, ones_v,
         p_s, acc_s, dacc_s, sem_g, sem_s) = scr
        cid = lax.axis_index("c")
        sid = lax.axis_index("s")
        wid = cid * NSUB + sid
        e0 = wid * EDGES_PER_TILE

        def load_idx(src_v, dst_v, g):
            base = e0 + g * CHUNK_E
            pltpu.sync_copy(src_hbm.at[pl.ds(base, CHUNK_E)], src_v)
            pltpu.sync_copy(dst_hbm.at[pl.ds(base, CHUNK_E)], dst_v)

        def fire_gather(src_v, rows_v):
            pltpu.async_copy(p_s.at[src_v], rows_v, sem_g)

        def load_and_fire(src_v, dst_v, rows_v, g):
            load_idx(src_v, dst_v, g)
            fire_gather(src_v, rows_v)

        def drain_gather(src_v, rows_v):
            pltpu.make_async_copy(p_s.at[src_v], rows_v, sem_g).wait()

        def fire_scatter(rows_v, dst_v):
            pltpu.async_copy(rows_v, acc_s.at[dst_v], sem_s, add=True)
            if with_deg:
                pltpu.async_copy(ones_v, dacc_s.at[dst_v], sem_s, add=True)

        def drain_scatter(rows_v, dst_v):
            pltpu.make_async_copy(rows_v, acc_s.at[dst_v], sem_s).wait()
            if with_deg:
                pltpu.make_async_copy(ones_v, dacc_s.at[dst_v], sem_s).wait()

        load_idx(src_a, dst_a, 0)
        if with_deg:
            pltpu.sync_copy(ones_hbm, ones_v)

        z0 = sid * ZROWS
        pltpu.sync_copy(p_hbm.at[pl.ds(z0, ZROWS)], p_s.at[pl.ds(z0, ZROWS)])
        pltpu.sync_copy(zeros_hbm, acc_s.at[pl.ds(z0, ZROWS)])
        if with_deg:
            pltpu.sync_copy(zeros1_hbm, dacc_s.at[pl.ds(z0, ZROWS)])
        plsc.subcore_barrier()
        fire_gather(src_a, rows_a)

        def step(g2, carry):
            c = g2 * 2
            drain_gather(src_a, rows_a)

            @pl.when(c > 0)
            def _():
                drain_scatter(rows_b, dst_b)

            load_and_fire(src_b, dst_b, rows_b, c + 1)
            fire_scatter(rows_a, dst_a)
            drain_gather(src_b, rows_b)
            drain_scatter(rows_a, dst_a)

            @pl.when(c + 2 < N_CHUNKS)
            def _():
                load_and_fire(src_a, dst_a, rows_a, c + 2)

            fire_scatter(rows_b, dst_b)
            return carry

        lax.fori_loop(0, N_CHUNKS // 2, step, 0)
        drain_scatter(rows_b, dst_b)
        plsc.subcore_barrier()

        pltpu.sync_copy(acc_s.at[pl.ds(z0, ZROWS)],
                        acc_out.at[cid, pl.ds(z0, ZROWS)])
        if with_deg:
            pltpu.sync_copy(dacc_s.at[pl.ds(z0, ZROWS)],
                            deg_out.at[cid, pl.ds(z0, ZROWS)])
        return None

    return pl.kernel(body, out_type=out_type, mesh=mesh,
                     scratch_types=scratch_types,
                     compiler_params=pltpu.CompilerParams(
                         use_tc_tiling_on_sc=False))


_seg_sum_deg = _make_seg_sum(True)
_seg_sum = _make_seg_sum(False)


NP = ACC_ROWS // 8
BP = NP // 5
BNODE = BP * 8


def _dotT(a, w):
    return lax.dot_general(a, w, (((1,), (1,)), ((), ())),
                           preferred_element_type=jnp.float32)


def _stage1_body(x_ref, wl_ref, wr_ref, p_ref, q_ref):
    xb = x_ref[...]
    wl = wl_ref[...]
    wr = wr_ref[...]
    p_ref[...] = jnp.concatenate(
        [_dotT(xb[:, a, :], wl) for a in range(8)], axis=1)
    q_ref[...] = jnp.concatenate(
        [_dotT(xb[:, a, :], wr) for a in range(8)], axis=1)


def _tc_stage1(x3, wl, wr):
    return pl.pallas_call(
        _stage1_body,
        grid=(5,),
        in_specs=[
            pl.BlockSpec((BP, 8, D_IN), lambda i: (i, 0, 0)),
            pl.BlockSpec((H, D_IN), lambda i: (0, 0)),
            pl.BlockSpec((H, D_IN), lambda i: (0, 0)),
        ],
        out_specs=[pl.BlockSpec((BP, 128), lambda i: (i, 0))] * 2,
        out_shape=[jax.ShapeDtypeStruct((NP, 128), jnp.float32)] * 2,
    )(x3, wl, wr)


_PK = pl.BlockSpec((BP, 128), lambda i: (i, 0))
_A0 = pl.BlockSpec((1, BP, 128), lambda i: (0, i, 0))
_A1 = pl.BlockSpec((1, BP, 128), lambda i: (1, i, 0))
_B = pl.BlockSpec((1, 128), lambda i: (0, 0))
_K = pl.BlockSpec((128, 128), lambda i: (0, 0))


_D0 = pl.BlockSpec((1, BP, 8), lambda i: (0, i, 0))
_D1 = pl.BlockSpec((1, BP, 8), lambda i: (1, i, 0))


def _mid_first_body(a0_ref, a1_ref, d0_ref, d1_ref, q_ref, b_ref, kl_ref,
                    kr_ref, k8_ref, p2_ref, q2_ref, rdeg_ref):
    dexp = jnp.dot(d0_ref[0] + d1_ref[0], k8_ref[...],
                   preferred_element_type=jnp.float32)
    rdeg = 1.0 / jnp.maximum(dexp, 1.0)
    h = jnp.maximum((a0_ref[0] + a1_ref[0]) * rdeg + b_ref[...] + q_ref[...],
                    0.0)
    p2_ref[...] = jnp.dot(h, kl_ref[...], preferred_element_type=jnp.float32)
    q2_ref[...] = jnp.dot(h, kr_ref[...], preferred_element_type=jnp.float32)
    rdeg_ref[...] = rdeg


def _tc_mid_first(accp, deg3, q, b, kl, kr, k8):
    return pl.pallas_call(
        _mid_first_body,
        grid=(5,),
        in_specs=[_A0, _A1, _D0, _D1, _PK, _B, _K, _K,
                  pl.BlockSpec((8, 128), lambda i: (0, 0))],
        out_specs=[_PK, _PK, _PK],
        out_shape=[jax.ShapeDtypeStruct((NP, 128), jnp.float32)] * 3,
    )(accp, accp, deg3, deg3, q, b, kl, kr, k8)


def _mid_body(a0_ref, a1_ref, rdeg_ref, q_ref, b_ref, kl_ref, kr_ref,
              p3_ref, q3_ref):
    h = jnp.maximum((a0_ref[0] + a1_ref[0]) * rdeg_ref[...]
                    + b_ref[...] + q_ref[...], 0.0)
    p3_ref[...] = jnp.dot(h, kl_ref[...], preferred_element_type=jnp.float32)
    q3_ref[...] = jnp.dot(h, kr_ref[...], preferred_element_type=jnp.float32)


def _tc_mid(accp, rdeg, q, b, kl, kr):
    return pl.pallas_call(
        _mid_body,
        grid=(5,),
        in_specs=[_A0, _A1, _PK, _PK, _B, _K, _K],
        out_specs=[_PK, _PK],
        out_shape=[jax.ShapeDtypeStruct((NP, 128), jnp.float32)] * 2,
    )(accp, accp, rdeg, q, b, kl, kr)


def _h3_body(a0_ref, a1_ref, rdeg_ref, q_ref, b_ref, h_ref):
    h_ref[...] = ((a0_ref[0] + a1_ref[0]) * rdeg_ref[...]
                  + b_ref[...] + q_ref[...])


def _tc_h3(accp, rdeg, q, b):
    return pl.pallas_call(
        _h3_body,
        grid=(5,),
        in_specs=[_A0, _A1, _PK, _PK, _B],
        out_specs=_PK,
        out_shape=jax.ShapeDtypeStruct((NP, 128), jnp.float32),
    )(accp, accp, rdeg, q, b)


def _fc_body(h_ref, wfc_ref, bfc_ref, out_ref):
    out_ref[...] = _dotT(h_ref[...], wfc_ref[...]) + bfc_ref[...]


def _tc_fc(hu, wfc, bfc):
    return pl.pallas_call(
        _fc_body,
        grid=(5,),
        in_specs=[pl.BlockSpec((N // 5, H), lambda i: (i, 0)),
                  pl.BlockSpec((R, H), lambda i: (0, 0)),
                  pl.BlockSpec((1, R), lambda i: (0, 0))],
        out_specs=pl.BlockSpec((N // 5, R), lambda i: (i, 0)),
        out_shape=jax.ShapeDtypeStruct((N, R), jnp.float32),
    )(hu, wfc, bfc)


@jax.jit
def kernel(x, edge_index, W1l, b1l, W1r, W2l, b2l, W2r, W3l, b3l, W3r, Wfc, bfc):
    pad = E_PAD - E
    src1d = jnp.concatenate([edge_index[0], jnp.zeros((pad,), jnp.int32)])
    dst1d = jnp.concatenate([edge_index[1], jnp.full((pad,), N, jnp.int32)])
    zeros_c = jnp.zeros((ZROWS, H), jnp.float32)
    zeros1_c = jnp.zeros((ZROWS,), jnp.float32)
    ones_c = jnp.ones((CHUNK_E,), jnp.float32)
    eye8 = jnp.eye(8, dtype=jnp.float32)
    k8 = jnp.kron(eye8, jnp.ones((1, H), jnp.float32))
    k2l, k2r = jnp.kron(eye8, W2l.T), jnp.kron(eye8, W2r.T)
    k3l, k3r = jnp.kron(eye8, W3l.T), jnp.kron(eye8, W3r.T)
    b1p = jnp.tile(b1l, 8).reshape(1, 128)
    b2p = jnp.tile(b2l, 8).reshape(1, 128)
    b3p = jnp.tile(b3l, 8).reshape(1, 128)

    p1, q1 = _tc_stage1(x.reshape(N // 8, 8, D_IN), W1l, W1r)
    acc1, deg = _seg_sum_deg(p1.reshape(ACC_ROWS, H), src1d, dst1d,
                             zeros_c, zeros1_c, ones_c)
    p2, q2, rdeg = _tc_mid_first(acc1.reshape(NCORES, NP, 128),
                                 deg.reshape(NCORES, NP, 8),
                                 q1, b1p, k2l, k2r, k8)
    acc2 = _seg_sum(p2.reshape(ACC_ROWS, H), src1d, dst1d,
                    zeros_c, zeros1_c, ones_c)[0]
    p3, q3 = _tc_mid(acc2.reshape(NCORES, NP, 128), rdeg, q2, b2p, k3l, k3r)
    acc3 = _seg_sum(p3.reshape(ACC_ROWS, H), src1d, dst1d,
                    zeros_c, zeros1_c, ones_c)[0]
    h3 = _tc_h3(acc3.reshape(NCORES, NP, 128), rdeg, q3, b3p)
    return _tc_fc(h3.reshape(ACC_ROWS, H), Wfc, bfc.reshape(1, R))
